# baseline (speedup 1.0000x reference)
import functools

import jax
import jax.numpy as jnp
from jax import lax
from jax.experimental import pallas as pl
from jax.experimental.pallas import tpu as pltpu
from jax.experimental.pallas import tpu_sc as plsc

N = 10000
NPAD = 10240
CH = NPAD // 16
RPT = 160
TROWS = 16 * RPT
TPAD = TROWS * 128
BM = 400
HID = 32


def _tc_body(adj, nf, w1s, w2s, w1d, w2d, win, binr, wout, boutr, ein, eout):
    KC = 400
    a = adj[...]
    f = nf[...]
    acc = jnp.zeros((a.shape[0], 1), jnp.float32)
    comp = jnp.zeros((a.shape[0], 1), jnp.float32)
    for k in range(N // KC):
        p = jnp.dot(a[:, k * KC:(k + 1) * KC], f[k * KC:(k + 1) * KC, :],
                    preferred_element_type=jnp.float32)
        y = p - comp
        t = acc + y
        comp = (t - acc) - y
        acc = t
    agg = acc
    ts = jnp.tanh(agg * w1s[...])
    td = jnp.tanh(agg * w1d[...])
    eout[...] = jnp.dot(jnp.dot(ts, w2s[...], preferred_element_type=jnp.float32),
                        wout[...], preferred_element_type=jnp.float32) + boutr[...]
    ein[...] = jnp.dot(jnp.dot(td, w2d[...], preferred_element_type=jnp.float32),
                       win[...], preferred_element_type=jnp.float32) + binr[...]


def _tc_scalars(adjm, nf, w1s, w2s, w1d, w2d, win, binr, wout, boutr):
    full = lambda shape: pl.BlockSpec(shape, lambda i: (0, 0))
    return pl.pallas_call(
        _tc_body,
        grid=(N // BM,),
        in_specs=[
            pl.BlockSpec((BM, N), lambda i: (i, 0)),
            full((N, 1)),
            full((1, HID)), full((HID, 24)),
            full((1, HID)), full((HID, 24)),
            full((24, 1)), full((1, 1)),
            full((24, 1)), full((1, 1)),
        ],
        out_specs=[
            pl.BlockSpec((BM, 1), lambda i: (i, 0)),
            pl.BlockSpec((BM, 1), lambda i: (i, 0)),
        ],
        out_shape=[
            jax.ShapeDtypeStruct((N, 1), jnp.float32),
            jax.ShapeDtypeStruct((N, 1), jnp.float32),
        ],
    )(adjm, nf, w1s, w2s, w1d, w2d, win, binr, wout, boutr)


def _sc_body(tripd, tripo, einp, eoutp, outd, outo,
             stage_d, stage_o, part_d, part_o,
             tbuf_d, tbuf_o, hist_d, hist_o, tmp2d_d, tmp2d_o,
             lp_d, lp_o, rk_d, rk_o, vals_d, vals_o,
             fill_d, fill_o, oidx_d, oidx_o, pvec, pall, e0d, e0o, sem):
    w = lax.axis_index("s")
    base_v = w * CH
    lane = lax.iota(jnp.int32, 16)

    z16 = jnp.zeros((16,), jnp.int32)
    for i in range(NPAD // 16):
        hist_d[pl.ds(i * 16, 16)] = z16
        hist_o[pl.ds(i * 16, 16)] = z16
    pltpu.sync_copy(einp.at[pl.ds(base_v, CH)], vals_d)
    pltpu.sync_copy(eoutp.at[pl.ds(base_v, CH)], vals_o)
    pltpu.sync_copy(einp.at[pl.ds(0, 16)], e0d)
    pltpu.sync_copy(eoutp.at[pl.ds(0, 16)], e0o)
    zidx = jnp.zeros((16,), jnp.int32)
    f_d = plsc.load_gather(e0d, [zidx])
    f_o = plsc.load_gather(e0o, [zidx])
    for k in range(CH // 16):
        fill_d[pl.ds(k * 16, 16)] = f_d
        fill_o[pl.ds(k * 16, 16)] = f_o
    pltpu.sync_copy(fill_d, outd.at[pl.ds(base_v, CH)])
    pltpu.sync_copy(fill_o, outo.at[pl.ds(base_v, CH)])

    pltpu.sync_copy(tripd.at[pl.ds(w * RPT, RPT)], tbuf_d)
    pltpu.sync_copy(tripo.at[pl.ds(w * RPT, RPT)], tbuf_o)
    one16 = jnp.ones((16,), jnp.int32)

    def p1(j, c):
        for g in range(128 // 16):
            plsc.store_scatter(hist_d, [tbuf_d[j, pl.ds(g * 16, 16)]], one16)
            plsc.store_scatter(hist_o, [tbuf_o[j, pl.ds(g * 16, 16)]], one16)
        return c

    lax.fori_loop(0, RPT, p1, 0)

    da = pltpu.async_copy(hist_d, stage_d.at[w], sem)
    db = pltpu.async_copy(hist_o, stage_o.at[w], sem)
    da.wait()
    db.wait()
    plsc.subcore_barrier()

    pltpu.sync_copy(stage_d.at[:, pl.ds(base_v, CH)], tmp2d_d)
    pltpu.sync_copy(stage_o.at[:, pl.ds(base_v, CH)], tmp2d_o)
    cd = jnp.int32(0)
    co = jnp.int32(0)
    for k in range(CH // 16):
        gidx = base_v + k * 16 + lane
        sd = tmp2d_d[0, pl.ds(k * 16, 16)]
        so = tmp2d_o[0, pl.ds(k * 16, 16)]
        for t in range(1, 16):
            sd = sd + tmp2d_d[t, pl.ds(k * 16, 16)]
            so = so + tmp2d_o[t, pl.ds(k * 16, 16)]
        pd = jnp.where((sd > 0) & (gidx < N), 1, 0)
        po = jnp.where((so > 0) & (gidx < N), 1, 0)
        rk_d[pl.ds(k * 16, 16)] = cd + plsc.cumsum(pd) - pd
        rk_o[pl.ds(k * 16, 16)] = co + plsc.cumsum(po) - po
        lp_d[pl.ds(k * 16, 16)] = pd
        lp_o[pl.ds(k * 16, 16)] = po
        cd = cd + jnp.sum(pd)
        co = co + jnp.sum(po)
    pvec[...] = jnp.where(lane == 0, cd, 0)
    pltpu.sync_copy(pvec, part_d.at[pl.ds(w * 16, 16)])
    pvec[...] = jnp.where(lane == 0, co, 0)
    pltpu.sync_copy(pvec, part_o.at[pl.ds(w * 16, 16)])
    plsc.subcore_barrier()

    pltpu.sync_copy(part_d, pall)
    accd = jnp.zeros((16,), jnp.int32)
    for k in range(16):
        accd = jnp.where(lane == k, jnp.sum(pall[pl.ds(k * 16, 16)]), accd)
    pltpu.sync_copy(part_o, pall)
    acco = jnp.zeros((16,), jnp.int32)
    for k in range(16):
        acco = jnp.where(lane == k, jnp.sum(pall[pl.ds(k * 16, 16)]), acco)
    offd = jnp.sum(jnp.where(lane < w, accd, 0))
    offo = jnp.sum(jnp.where(lane < w, acco, 0))
    dump = N + 8 * w
    for k in range(CH // 16):
        r, c = (k * 16) // 128, (k * 16) % 128
        pd = lp_d[pl.ds(k * 16, 16)]
        po = lp_o[pl.ds(k * 16, 16)]
        oidx_d[r, pl.ds(c, 16)] = jnp.where(pd > 0, offd + rk_d[pl.ds(k * 16, 16)], dump)
        oidx_o[r, pl.ds(c, 16)] = jnp.where(po > 0, offo + rk_o[pl.ds(k * 16, 16)], dump)
    plsc.subcore_barrier()
    for j in range(CH // 128):
        pltpu.sync_copy(vals_d.at[pl.ds(j * 128, 128)], outd.at[oidx_d.at[j]])
        pltpu.sync_copy(vals_o.at[pl.ds(j * 128, 128)], outo.at[oidx_o.at[j]])


_sc_unique_gather = functools.partial(
    pl.kernel,
    out_type=[jax.ShapeDtypeStruct((NPAD,), jnp.float32),
              jax.ShapeDtypeStruct((NPAD,), jnp.float32)],
    mesh=plsc.VectorSubcoreMesh(core_axis_name="c", subcore_axis_name="s",
                                num_cores=1),
    compiler_params=pltpu.CompilerParams(needs_layout_passes=False),
    scratch_types=[
        pltpu.VMEM_SHARED((16, NPAD), jnp.int32),
        pltpu.VMEM_SHARED((16, NPAD), jnp.int32),
        pltpu.VMEM_SHARED((256,), jnp.int32),
        pltpu.VMEM_SHARED((256,), jnp.int32),
        pltpu.VMEM((RPT, 128), jnp.int32),
        pltpu.VMEM((RPT, 128), jnp.int32),
        pltpu.VMEM((NPAD,), jnp.int32),
        pltpu.VMEM((NPAD,), jnp.int32),
        pltpu.VMEM((16, CH), jnp.int32),
        pltpu.VMEM((16, CH), jnp.int32),
        pltpu.VMEM((CH,), jnp.int32),
        pltpu.VMEM((CH,), jnp.int32),
        pltpu.VMEM((CH,), jnp.int32),
        pltpu.VMEM((CH,), jnp.int32),
        pltpu.VMEM((CH,), jnp.float32),
        pltpu.VMEM((CH,), jnp.float32),
        pltpu.VMEM((CH,), jnp.float32),
        pltpu.VMEM((CH,), jnp.float32),
        pltpu.VMEM((CH // 128, 128), jnp.int32),
        pltpu.VMEM((CH // 128, 128), jnp.int32),
        pltpu.VMEM((16,), jnp.int32),
        pltpu.VMEM((256,), jnp.int32),
        pltpu.VMEM((16,), jnp.float32),
        pltpu.VMEM((16,), jnp.float32),
        pltpu.SemaphoreType.DMA,
    ],
)(_sc_body)


def kernel(adjm, node_feats, trip_od, W1s, W2s, W1d, W2d, Win, bin_, Wout, bout):
    ein, eout = _tc_scalars(
        adjm, node_feats, W1s, W2s, W1d, W2d,
        Win, bin_.reshape(1, 1), Wout, bout.reshape(1, 1))
    pad = jnp.full((TPAD - trip_od.shape[0],), N + 16, dtype=jnp.int32)
    tripd = jnp.concatenate([trip_od[:, 1], pad]).reshape(TROWS, 128)
    tripo = jnp.concatenate([trip_od[:, 0], pad]).reshape(TROWS, 128)
    einp = jnp.pad(ein.reshape(-1), (0, NPAD - N))
    eoutp = jnp.pad(eout.reshape(-1), (0, NPAD - N))
    outd, outo = _sc_unique_gather(tripd, tripo, einp, eoutp)
    return jnp.stack([outd[:N], outo[:N]], axis=1)


# speedup vs baseline: 1.0563x; 1.0563x over previous
import functools

import jax
import jax.numpy as jnp
from jax import lax
from jax.experimental import pallas as pl
from jax.experimental.pallas import tpu as pltpu
from jax.experimental.pallas import tpu_sc as plsc

N = 10000
NPAD = 10240
CH = NPAD // 16
RPT = 160
TROWS = 16 * RPT
TPAD = TROWS * 128
BM = 400
HID = 32


def _tc_body(adj, nf, w1s, w2s, w1d, w2d, win, binr, wout, boutr, ein, eout):
    KC = 400
    a = adj[...]
    f = nf[...]
    acc = jnp.zeros((a.shape[0], 1), jnp.float32)
    comp = jnp.zeros((a.shape[0], 1), jnp.float32)
    for k in range(N // KC):
        p = jnp.dot(a[:, k * KC:(k + 1) * KC], f[k * KC:(k + 1) * KC, :],
                    preferred_element_type=jnp.float32)
        y = p - comp
        t = acc + y
        comp = (t - acc) - y
        acc = t
    agg = acc
    ts = jnp.tanh(agg * w1s[...])
    td = jnp.tanh(agg * w1d[...])
    eout[...] = jnp.dot(jnp.dot(ts, w2s[...], preferred_element_type=jnp.float32),
                        wout[...], preferred_element_type=jnp.float32) + boutr[...]
    ein[...] = jnp.dot(jnp.dot(td, w2d[...], preferred_element_type=jnp.float32),
                       win[...], preferred_element_type=jnp.float32) + binr[...]


def _tc_scalars(adjm, nf, w1s, w2s, w1d, w2d, win, binr, wout, boutr):
    full = lambda shape: pl.BlockSpec(shape, lambda i: (0, 0))
    return pl.pallas_call(
        _tc_body,
        grid=(N // BM,),
        in_specs=[
            pl.BlockSpec((BM, N), lambda i: (i, 0)),
            full((N, 1)),
            full((1, HID)), full((HID, 24)),
            full((1, HID)), full((HID, 24)),
            full((24, 1)), full((1, 1)),
            full((24, 1)), full((1, 1)),
        ],
        out_specs=[
            pl.BlockSpec((BM, 1), lambda i: (i, 0)),
            pl.BlockSpec((BM, 1), lambda i: (i, 0)),
        ],
        out_shape=[
            jax.ShapeDtypeStruct((N, 1), jnp.float32),
            jax.ShapeDtypeStruct((N, 1), jnp.float32),
        ],
    )(adjm, nf, w1s, w2s, w1d, w2d, win, binr, wout, boutr)


def _sca_body(tripd, tripo, oidxd_h, oidxo_h,
              stage_d, stage_o, part_d, part_o,
              tbuf_d, tbuf_o, hist_d, hist_o, tmp2d_d, tmp2d_o,
              lp_d, lp_o, rk_d, rk_o,
              oidx_d, oidx_o, pvec, pall, sem):
    w = lax.axis_index("s")
    base_v = w * CH
    lane = lax.iota(jnp.int32, 16)

    z16 = jnp.zeros((16,), jnp.int32)
    for i in range(NPAD // 16):
        hist_d[pl.ds(i * 16, 16)] = z16
        hist_o[pl.ds(i * 16, 16)] = z16

    pltpu.sync_copy(tripd.at[pl.ds(w * RPT, RPT)], tbuf_d)
    pltpu.sync_copy(tripo.at[pl.ds(w * RPT, RPT)], tbuf_o)
    one16 = jnp.ones((16,), jnp.int32)

    def p1(j, c):
        for g in range(128 // 16):
            plsc.store_scatter(hist_d, [tbuf_d[j, pl.ds(g * 16, 16)]], one16)
            plsc.store_scatter(hist_o, [tbuf_o[j, pl.ds(g * 16, 16)]], one16)
        return c

    lax.fori_loop(0, RPT, p1, 0)

    da = pltpu.async_copy(hist_d, stage_d.at[w], sem)
    db = pltpu.async_copy(hist_o, stage_o.at[w], sem)
    da.wait()
    db.wait()
    plsc.subcore_barrier()

    pltpu.sync_copy(stage_d.at[:, pl.ds(base_v, CH)], tmp2d_d)
    pltpu.sync_copy(stage_o.at[:, pl.ds(base_v, CH)], tmp2d_o)
    cd = jnp.int32(0)
    co = jnp.int32(0)
    for k in range(CH // 16):
        gidx = base_v + k * 16 + lane
        sd = tmp2d_d[0, pl.ds(k * 16, 16)]
        so = tmp2d_o[0, pl.ds(k * 16, 16)]
        for t in range(1, 16):
            sd = sd + tmp2d_d[t, pl.ds(k * 16, 16)]
            so = so + tmp2d_o[t, pl.ds(k * 16, 16)]
        pd = jnp.where((sd > 0) & (gidx < N), 1, 0)
        po = jnp.where((so > 0) & (gidx < N), 1, 0)
        rk_d[pl.ds(k * 16, 16)] = cd + plsc.cumsum(pd) - pd
        rk_o[pl.ds(k * 16, 16)] = co + plsc.cumsum(po) - po
        lp_d[pl.ds(k * 16, 16)] = pd
        lp_o[pl.ds(k * 16, 16)] = po
        cd = cd + jnp.sum(pd)
        co = co + jnp.sum(po)
    pvec[...] = jnp.where(lane == 0, cd, 0)
    pltpu.sync_copy(pvec, part_d.at[pl.ds(w * 16, 16)])
    pvec[...] = jnp.where(lane == 0, co, 0)
    pltpu.sync_copy(pvec, part_o.at[pl.ds(w * 16, 16)])
    plsc.subcore_barrier()

    pltpu.sync_copy(part_d, pall)
    accd = jnp.zeros((16,), jnp.int32)
    for k in range(16):
        accd = jnp.where(lane == k, jnp.sum(pall[pl.ds(k * 16, 16)]), accd)
    pltpu.sync_copy(part_o, pall)
    acco = jnp.zeros((16,), jnp.int32)
    for k in range(16):
        acco = jnp.where(lane == k, jnp.sum(pall[pl.ds(k * 16, 16)]), acco)
    offd = jnp.sum(jnp.where(lane < w, accd, 0))
    offo = jnp.sum(jnp.where(lane < w, acco, 0))
    dump = jnp.full((16,), N + 8 * w, jnp.int32)
    for k in range(CH // 16):
        r, c = (k * 16) // 128, (k * 16) % 128
        pd = lp_d[pl.ds(k * 16, 16)]
        po = lp_o[pl.ds(k * 16, 16)]
        oidx_d[r, pl.ds(c, 16)] = jnp.where(pd > 0, offd + rk_d[pl.ds(k * 16, 16)], dump)
        oidx_o[r, pl.ds(c, 16)] = jnp.where(po > 0, offo + rk_o[pl.ds(k * 16, 16)], dump)
    for k in range(CH // 16, CH // 16 + 3 * 8):
        r, c = (k * 16) // 128, (k * 16) % 128
        oidx_d[r, pl.ds(c, 16)] = dump
        oidx_o[r, pl.ds(c, 16)] = dump
    pltpu.sync_copy(oidx_d, oidxd_h.at[pl.ds(w * 8, 8)])
    pltpu.sync_copy(oidx_o, oidxo_h.at[pl.ds(w * 8, 8)])


def _scb_body(oidxd_h, oidxo_h, einp, eoutp, outd, outo,
              vals_d, vals_o, fill_d, fill_o, oidx_d, oidx_o, e0d, e0o):
    w = lax.axis_index("s")
    base_v = w * CH

    pltpu.sync_copy(einp.at[pl.ds(base_v, CH)], vals_d)
    pltpu.sync_copy(eoutp.at[pl.ds(base_v, CH)], vals_o)
    pltpu.sync_copy(einp.at[pl.ds(0, 16)], e0d)
    pltpu.sync_copy(eoutp.at[pl.ds(0, 16)], e0o)
    pltpu.sync_copy(oidxd_h.at[pl.ds(w * 8, 8)], oidx_d)
    pltpu.sync_copy(oidxo_h.at[pl.ds(w * 8, 8)], oidx_o)
    zidx = jnp.zeros((16,), jnp.int32)
    f_d = plsc.load_gather(e0d, [zidx])
    f_o = plsc.load_gather(e0o, [zidx])
    for k in range(CH // 16):
        fill_d[pl.ds(k * 16, 16)] = f_d
        fill_o[pl.ds(k * 16, 16)] = f_o
    pltpu.sync_copy(fill_d, outd.at[pl.ds(base_v, CH)])
    pltpu.sync_copy(fill_o, outo.at[pl.ds(base_v, CH)])
    plsc.subcore_barrier()
    for j in range(CH // 128):
        pltpu.sync_copy(vals_d.at[pl.ds(j * 128, 128)], outd.at[oidx_d.at[j]])
        pltpu.sync_copy(vals_o.at[pl.ds(j * 128, 128)], outo.at[oidx_o.at[j]])


_SC_MESH = plsc.VectorSubcoreMesh(core_axis_name="c", subcore_axis_name="s",
                                  num_cores=1)

_sc_ranks = functools.partial(
    pl.kernel,
    out_type=[jax.ShapeDtypeStruct((128, 128), jnp.int32),
              jax.ShapeDtypeStruct((128, 128), jnp.int32)],
    mesh=_SC_MESH,
    compiler_params=pltpu.CompilerParams(needs_layout_passes=False),
    scratch_types=[
        pltpu.VMEM_SHARED((16, NPAD), jnp.int32),
        pltpu.VMEM_SHARED((16, NPAD), jnp.int32),
        pltpu.VMEM_SHARED((256,), jnp.int32),
        pltpu.VMEM_SHARED((256,), jnp.int32),
        pltpu.VMEM((RPT, 128), jnp.int32),
        pltpu.VMEM((RPT, 128), jnp.int32),
        pltpu.VMEM((NPAD,), jnp.int32),
        pltpu.VMEM((NPAD,), jnp.int32),
        pltpu.VMEM((16, CH), jnp.int32),
        pltpu.VMEM((16, CH), jnp.int32),
        pltpu.VMEM((CH,), jnp.int32),
        pltpu.VMEM((CH,), jnp.int32),
        pltpu.VMEM((CH,), jnp.int32),
        pltpu.VMEM((CH,), jnp.int32),
        pltpu.VMEM((8, 128), jnp.int32),
        pltpu.VMEM((8, 128), jnp.int32),
        pltpu.VMEM((16,), jnp.int32),
        pltpu.VMEM((256,), jnp.int32),
        pltpu.SemaphoreType.DMA,
    ],
)(_sca_body)

_sc_scatter = functools.partial(
    pl.kernel,
    out_type=[jax.ShapeDtypeStruct((NPAD,), jnp.float32),
              jax.ShapeDtypeStruct((NPAD,), jnp.float32)],
    mesh=_SC_MESH,
    compiler_params=pltpu.CompilerParams(needs_layout_passes=False),
    scratch_types=[
        pltpu.VMEM((CH,), jnp.float32),
        pltpu.VMEM((CH,), jnp.float32),
        pltpu.VMEM((CH,), jnp.float32),
        pltpu.VMEM((CH,), jnp.float32),
        pltpu.VMEM((8, 128), jnp.int32),
        pltpu.VMEM((8, 128), jnp.int32),
        pltpu.VMEM((16,), jnp.float32),
        pltpu.VMEM((16,), jnp.float32),
    ],
)(_scb_body)


def kernel(adjm, node_feats, trip_od, W1s, W2s, W1d, W2d, Win, bin_, Wout, bout):
    pad = jnp.full((TPAD - trip_od.shape[0],), N + 16, dtype=jnp.int32)
    tripd = jnp.concatenate([trip_od[:, 1], pad]).reshape(TROWS, 128)
    tripo = jnp.concatenate([trip_od[:, 0], pad]).reshape(TROWS, 128)
    oidxd_h, oidxo_h = _sc_ranks(tripd, tripo)
    ein, eout = _tc_scalars(
        adjm, node_feats, W1s, W2s, W1d, W2d,
        Win, bin_.reshape(1, 1), Wout, bout.reshape(1, 1))
    einp = jnp.pad(ein.reshape(-1), (0, NPAD - N))
    eoutp = jnp.pad(eout.reshape(-1), (0, NPAD - N))
    outd, outo = _sc_scatter(oidxd_h, oidxo_h, einp, eoutp)
    return jnp.stack([outd[:N], outo[:N]], axis=1)
